# baseline (speedup 1.0000x reference)
import functools

import jax
import jax.numpy as jnp
from jax import lax
from jax.experimental import pallas as pl
from jax.experimental.pallas import tpu as pltpu, tpu_sc as plsc

N = 50000
E = 800000
G = 32

NP = 50176
Bn = 3136
NB = NP // Bn
Ep = 819200
S = 200
NIT0 = 100
NIT1 = 28
EW0 = 2 * S * NIT0
EW1 = 2 * S * NIT1
EWMX = EW0
EPA = 16 * EW0 + 15 * EW1 + EWMX
APT = NP // 16
NPP = NP // 8
PBn = Bn // 8


@functools.cache
def _mesh():
    return plsc.VectorSubcoreMesh(core_axis_name="c", subcore_axis_name="s",
                                  num_cores=2, num_subcores=16)


@functools.cache
def _sc_pass(nchunks, w, gather):
    scratch = [
        pltpu.VMEM((S,), jnp.int32),
        pltpu.VMEM((S,), jnp.int32),
    ]
    if gather:
        scratch.append(pltpu.VMEM((EWMX,), jnp.int32))
    scratch += [
        pltpu.VMEM((S, w), jnp.float32),
        pltpu.VMEM((S, w), jnp.float32),
        pltpu.VMEM_SHARED((NP, w), jnp.float32),
        pltpu.SemaphoreType.DMA,
        pltpu.SemaphoreType.DMA,
        pltpu.SemaphoreType.DMA,
        pltpu.SemaphoreType.DMA,
    ]

    @functools.partial(
        pl.kernel,
        out_type=jax.ShapeDtypeStruct((2, nchunks, NP, w), jnp.float32),
        mesh=_mesh(),
        compiler_params=pltpu.CompilerParams(use_tc_tiling_on_sc=False),
        scratch_types=scratch,
    )
    def sc_pass(*refs):
        if gather:
            tables = refs[:nchunks]
            (src_h, dst_h, zeros_h, out_h, idxd_a, idxd_b, idxs_all,
             rows_a, rows_b, acc, g_a, g_b, s_a, s_b) = refs[nchunks:]
        else:
            (dst_h, ones_h, zeros_h, out_h, idxd_a, idxd_b,
             rows_a, rows_b, acc, g_a, g_b, s_a, s_b) = refs
        cid = lax.axis_index("c")
        sid = lax.axis_index("s")
        wedge = pl.multiple_of(
            jnp.where(cid == 0, sid * EW0, 16 * EW0 + sid * EW1), 8)
        nit = jnp.where(cid == 0, NIT0, NIT1)
        ew = jnp.where(cid == 0, EW0, EW1)
        arow = pl.multiple_of(sid * APT, 8)

        if gather:
            pltpu.sync_copy(src_h.at[pl.ds(wedge, EWMX)], idxs_all)
        else:
            pltpu.sync_copy(ones_h, rows_a)
            pltpu.sync_copy(ones_h, rows_b)

        def load_d(i, k, buf):
            off = pl.multiple_of(wedge + (2 * i + k) * S, 8)
            pltpu.sync_copy(dst_h.at[pl.ds(off, S)], buf)

        def s_idx(i, k):
            return idxs_all.at[pl.ds(pl.multiple_of((2 * i + k) * S, 8), S)]

        def drain_s(rows, sem):
            pltpu.make_async_copy(rows, acc.at[idxd_a], sem).wait()

        for c in range(nchunks):
            pltpu.sync_copy(zeros_h.at[pl.ds(arow, APT)],
                            acc.at[pl.ds(arow, APT)])
            plsc.subcore_barrier()

            table = tables[c] if gather else None

            def it(i, carry, table=table):
                if gather:
                    @pl.when(i > 0)
                    def _():
                        drain_s(rows_a, s_a)
                    pltpu.async_copy(table.at[s_idx(i, 0)], rows_a, g_a)

                    @pl.when(i > 0)
                    def _():
                        drain_s(rows_b, s_b)
                    pltpu.async_copy(table.at[s_idx(i, 1)], rows_b, g_b)
                    load_d(i, 0, idxd_a)
                    load_d(i, 1, idxd_b)
                    pltpu.make_async_copy(table.at[s_idx(i, 0)],
                                          rows_a, g_a).wait()
                    pltpu.async_copy(rows_a, acc.at[idxd_a], s_a, add=True)
                    pltpu.make_async_copy(table.at[s_idx(i, 1)],
                                          rows_b, g_b).wait()
                    pltpu.async_copy(rows_b, acc.at[idxd_b], s_b, add=True)
                else:
                    @pl.when(i > 0)
                    def _():
                        drain_s(rows_a, s_a)
                        drain_s(rows_b, s_b)
                    load_d(i, 0, idxd_a)
                    load_d(i, 1, idxd_b)
                    pltpu.async_copy(rows_a, acc.at[idxd_a], s_a, add=True)
                    pltpu.async_copy(rows_b, acc.at[idxd_b], s_b, add=True)
                return carry

            lax.fori_loop(0, nit, it, 0)
            drain_s(rows_a, s_a)
            drain_s(rows_b, s_b)
            plsc.subcore_barrier()
            pltpu.sync_copy(acc.at[pl.ds(arow, APT)],
                            out_h.at[cid].at[c].at[pl.ds(arow, APT)])
            plsc.subcore_barrier()

    return sc_pass


def _dis_packed(dp):
    deg = dp[0, 0] + dp[1, 0] + 1.0
    return lax.rsqrt(deg)


def _tc_prep_body(dp_ref, xp_ref, xs_ref):
    xs_ref[...] = xp_ref[...] * _dis_packed(dp_ref[...])


def _tc_mid_body(xs_ref, ax_ref, dp_ref, w_ref, b_ref, *out_refs):
    disp = _dis_packed(dp_ref[...])
    a1p = (xs_ref[...] + ax_ref[0, 0] + ax_ref[1, 0]) * disp
    for c in range(7):
        h = jnp.dot(a1p, w_ref[c], preferred_element_type=jnp.float32)
        h = h + b_ref[c]
        h = h * jax.nn.sigmoid(h)
        out_refs[c][...] = h * disp


def _tc_final_body(t0, t1, t2, t3, t4, t5, t6, ah_ref, dp_ref, bt_ref,
                   w2_ref, b2_ref, wl1_ref, bl1_ref, wl2_ref, bl2_ref,
                   out_ref, sums, cnt):
    i = pl.program_id(0)

    @pl.when(i == 0)
    def _():
        sums[...] = jnp.zeros_like(sums)
        cnt[...] = jnp.zeros_like(cnt)

    disp = _dis_packed(dp_ref[...])
    ts = (t0, t1, t2, t3, t4, t5, t6)
    h2 = b2_ref[...]
    for c in range(7):
        p_c = (ts[c][...] + ah_ref[0, c] + ah_ref[1, c]) * disp
        h2 = h2 + jnp.dot(p_c, w2_ref[c], preferred_element_type=jnp.float32)
    h2 = h2 * jax.nn.sigmoid(h2)
    gids = lax.broadcasted_iota(jnp.int32, (G, PBn), 0)
    for j in range(8):
        ohj = (gids == bt_ref[0, j, :][None, :]).astype(jnp.float32)
        sums[...] += jnp.dot(ohj, h2[:, 200 * j:200 * (j + 1)],
                             preferred_element_type=jnp.float32)
        cnt[...] += jnp.broadcast_to(
            jnp.sum(ohj, axis=1, keepdims=True), (G, 128))

    @pl.when(i == NB - 1)
    def _():
        pooled = sums[...] / jnp.maximum(cnt[:, 0:1], 1.0)
        z = jnp.dot(pooled, wl1_ref[...],
                    preferred_element_type=jnp.float32) + bl1_ref[...]
        z = z * jax.nn.sigmoid(z)
        o = jnp.dot(z, wl2_ref[...],
                    preferred_element_type=jnp.float32) + bl2_ref[...]
        out_ref[...] = o


@functools.cache
def _tc_kernels(interpret=False):
    tc_prep = pl.pallas_call(
        _tc_prep_body,
        interpret=interpret,
        grid=(NB,),
        in_specs=[
            pl.BlockSpec((2, 1, PBn, 128), lambda i: (0, 0, i, 0)),
            pl.BlockSpec((PBn, 128), lambda i: (i, 0)),
        ],
        out_specs=pl.BlockSpec((PBn, 128), lambda i: (i, 0)),
        out_shape=jax.ShapeDtypeStruct((NPP, 128), jnp.float32),
    )
    tc_mid = pl.pallas_call(
        _tc_mid_body,
        interpret=interpret,
        grid=(NB,),
        in_specs=[
            pl.BlockSpec((PBn, 128), lambda i: (i, 0)),
            pl.BlockSpec((2, 1, PBn, 128), lambda i: (0, 0, i, 0)),
            pl.BlockSpec((2, 1, PBn, 128), lambda i: (0, 0, i, 0)),
            pl.BlockSpec((7, 128, 128), lambda i: (0, 0, 0)),
            pl.BlockSpec((7, 1, 128), lambda i: (0, 0, 0)),
        ],
        out_specs=[pl.BlockSpec((PBn, 128), lambda i: (i, 0)) for _ in range(7)],
        out_shape=[jax.ShapeDtypeStruct((NPP, 128), jnp.float32) for _ in range(7)],
    )
    tc_final = pl.pallas_call(
        _tc_final_body,
        interpret=interpret,
        grid=(NB,),
        in_specs=(
            [pl.BlockSpec((PBn, 128), lambda i: (i, 0)) for _ in range(7)] + [
                pl.BlockSpec((2, 7, PBn, 128), lambda i: (0, 0, i, 0)),
                pl.BlockSpec((2, 1, PBn, 128), lambda i: (0, 0, i, 0)),
                pl.BlockSpec((1, 8, PBn), lambda i: (i, 0, 0)),
                pl.BlockSpec((7, 128, 1600), lambda i: (0, 0, 0)),
                pl.BlockSpec((1, 1600), lambda i: (0, 0)),
                pl.BlockSpec((200, 100), lambda i: (0, 0)),
                pl.BlockSpec((1, 100), lambda i: (0, 0)),
                pl.BlockSpec((100, 128), lambda i: (0, 0)),
                pl.BlockSpec((1, 128), lambda i: (0, 0)),
            ]
        ),
        out_specs=pl.BlockSpec((G, 128), lambda i: (0, 0)),
        out_shape=jax.ShapeDtypeStruct((G, 128), jnp.float32),
        scratch_shapes=[
            pltpu.VMEM((G, 200), jnp.float32),
            pltpu.VMEM((G, 128), jnp.float32),
        ],
    )
    return tc_prep, tc_mid, tc_final


def kernel(x, edge_index, batch, W1, b1, W2, b2, Wl1, bl1, Wl2, bl2):
    f32 = jnp.float32
    src = edge_index[0]
    dst = edge_index[1]
    epad = jnp.full((EPA - E,), N, jnp.int32)
    srcp = jnp.concatenate([src, epad])
    dstp = jnp.concatenate([dst, epad])
    xpad = jnp.zeros((NP, 16), f32).at[:N, :3].set(
        x.astype(f32)).reshape(NPP, 128)
    zeros16 = jnp.zeros((NP, 16), f32)
    onesv = jnp.ones((S, 16), f32)
    batchp = jnp.concatenate(
        [batch, jnp.full((NP - N,), G, jnp.int32)]).reshape(
            NB, PBn, 8).transpose(0, 2, 1)
    W1p = jnp.zeros((16, 112), f32).at[:3, :100].set(W1)
    b1p = jnp.zeros((112,), f32).at[:100].set(b1)
    eye8 = jnp.eye(8, dtype=f32)
    W1big = jnp.stack([
        jnp.einsum('ab,km->akbm', eye8,
                   W1p[:, 16 * c:16 * (c + 1)]).reshape(128, 128)
        for c in range(7)])
    b1big = jnp.stack([jnp.tile(b1p[16 * c:16 * (c + 1)], 8).reshape(1, 128)
                       for c in range(7)])
    W2p = jnp.zeros((112, 200), f32).at[:100, :].set(W2)
    W2big = jnp.stack([
        jnp.einsum('ab,km->akbm', eye8,
                   W2p[16 * c:16 * (c + 1), :]).reshape(128, 1600)
        for c in range(7)])
    b2big = jnp.tile(b2, 8).reshape(1, 1600)
    bl1r = bl1.reshape(1, 100)
    Wl2p = jnp.zeros((100, 128), f32).at[:, 0:1].set(Wl2)
    bl2p = jnp.broadcast_to(bl2.reshape(1, 1), (1, 128))

    tc_prep, tc_mid, tc_final = _tc_kernels()
    degpart = _sc_pass(1, 16, False)(dstp, onesv, zeros16)
    degp = degpart.reshape(2, 1, NPP, 128)
    xs_p = tc_prep(degp, xpad)
    xs = xs_p.reshape(NP, 16)
    aggx = _sc_pass(1, 16, True)(xs, srcp, dstp, zeros16)
    ts_p = tc_mid(xs_p, aggx.reshape(2, 1, NPP, 128), degp, W1big, b1big)
    ts = [t.reshape(NP, 16) for t in ts_p]
    aggh = _sc_pass(7, 16, True)(*ts, srcp, dstp, zeros16)
    out = tc_final(*ts_p, aggh.reshape(2, 7, NPP, 128), degp, batchp,
                   W2big, b2big, Wl1, bl1r, Wl2p, bl2p)
    return out[:, 0]


# speedup vs baseline: 1.0556x; 1.0556x over previous
import functools

import jax
import jax.numpy as jnp
from jax import lax
from jax.experimental import pallas as pl
from jax.experimental.pallas import tpu as pltpu, tpu_sc as plsc

N = 50000
E = 800000
G = 32

NP = 50176
Bn = 3136
NB = NP // Bn
Ep = 819200
S = 400
NIT0 = 50
NIT1 = 14
EW0 = 2 * S * NIT0
EW1 = 2 * S * NIT1
EWMX = EW0
EPA = 16 * EW0 + 15 * EW1 + EWMX
APT = NP // 16
NPP = NP // 8
PBn = Bn // 8


@functools.cache
def _mesh():
    return plsc.VectorSubcoreMesh(core_axis_name="c", subcore_axis_name="s",
                                  num_cores=2, num_subcores=16)


@functools.cache
def _sc_pass(nchunks, w, gather):
    scratch = [
        pltpu.VMEM((S,), jnp.int32),
        pltpu.VMEM((S,), jnp.int32),
    ]
    if gather:
        scratch.append(pltpu.VMEM((EWMX,), jnp.int32))
    scratch += [
        pltpu.VMEM((S, w), jnp.float32),
        pltpu.VMEM((S, w), jnp.float32),
        pltpu.VMEM_SHARED((NP, w), jnp.float32),
        pltpu.SemaphoreType.DMA,
        pltpu.SemaphoreType.DMA,
        pltpu.SemaphoreType.DMA,
        pltpu.SemaphoreType.DMA,
    ]

    @functools.partial(
        pl.kernel,
        out_type=jax.ShapeDtypeStruct((2, nchunks, NP, w), jnp.float32),
        mesh=_mesh(),
        compiler_params=pltpu.CompilerParams(use_tc_tiling_on_sc=False),
        scratch_types=scratch,
    )
    def sc_pass(*refs):
        if gather:
            tables = refs[:nchunks]
            (src_h, dst_h, zeros_h, out_h, idxd_a, idxd_b, idxs_all,
             rows_a, rows_b, acc, g_a, g_b, s_a, s_b) = refs[nchunks:]
        else:
            (dst_h, ones_h, zeros_h, out_h, idxd_a, idxd_b,
             rows_a, rows_b, acc, g_a, g_b, s_a, s_b) = refs
        cid = lax.axis_index("c")
        sid = lax.axis_index("s")
        wedge = pl.multiple_of(
            jnp.where(cid == 0, sid * EW0, 16 * EW0 + sid * EW1), 8)
        nit = jnp.where(cid == 0, NIT0, NIT1)
        ew = jnp.where(cid == 0, EW0, EW1)
        arow = pl.multiple_of(sid * APT, 8)

        if gather:
            pltpu.sync_copy(src_h.at[pl.ds(wedge, EWMX)], idxs_all)
        else:
            pltpu.sync_copy(ones_h, rows_a)
            pltpu.sync_copy(ones_h, rows_b)

        def load_d(i, k, buf):
            off = pl.multiple_of(wedge + (2 * i + k) * S, 8)
            pltpu.sync_copy(dst_h.at[pl.ds(off, S)], buf)

        def s_idx(i, k):
            return idxs_all.at[pl.ds(pl.multiple_of((2 * i + k) * S, 8), S)]

        def drain_s(rows, sem):
            pltpu.make_async_copy(rows, acc.at[idxd_a], sem).wait()

        for c in range(nchunks):
            pltpu.sync_copy(zeros_h.at[pl.ds(arow, APT)],
                            acc.at[pl.ds(arow, APT)])
            plsc.subcore_barrier()

            table = tables[c] if gather else None

            def it(i, carry, table=table):
                if gather:
                    @pl.when(i > 0)
                    def _():
                        drain_s(rows_a, s_a)
                    pltpu.async_copy(table.at[s_idx(i, 0)], rows_a, g_a)

                    @pl.when(i > 0)
                    def _():
                        drain_s(rows_b, s_b)
                    pltpu.async_copy(table.at[s_idx(i, 1)], rows_b, g_b)
                    load_d(i, 0, idxd_a)
                    load_d(i, 1, idxd_b)
                    pltpu.make_async_copy(table.at[s_idx(i, 0)],
                                          rows_a, g_a).wait()
                    pltpu.async_copy(rows_a, acc.at[idxd_a], s_a, add=True)
                    pltpu.make_async_copy(table.at[s_idx(i, 1)],
                                          rows_b, g_b).wait()
                    pltpu.async_copy(rows_b, acc.at[idxd_b], s_b, add=True)
                else:
                    @pl.when(i > 0)
                    def _():
                        drain_s(rows_a, s_a)
                        drain_s(rows_b, s_b)
                    load_d(i, 0, idxd_a)
                    load_d(i, 1, idxd_b)
                    pltpu.async_copy(rows_a, acc.at[idxd_a], s_a, add=True)
                    pltpu.async_copy(rows_b, acc.at[idxd_b], s_b, add=True)
                return carry

            lax.fori_loop(0, nit, it, 0)
            drain_s(rows_a, s_a)
            drain_s(rows_b, s_b)
            plsc.subcore_barrier()
            pltpu.sync_copy(acc.at[pl.ds(arow, APT)],
                            out_h.at[cid].at[c].at[pl.ds(arow, APT)])
            plsc.subcore_barrier()

    return sc_pass


def _dis_packed(dp):
    deg = dp[0, 0] + dp[1, 0] + 1.0
    return lax.rsqrt(deg)


def _tc_prep_body(dp_ref, xp_ref, xs_ref):
    xs_ref[...] = xp_ref[...] * _dis_packed(dp_ref[...])


def _tc_mid_body(xs_ref, ax_ref, dp_ref, w_ref, b_ref, *out_refs):
    disp = _dis_packed(dp_ref[...])
    a1p = (xs_ref[...] + ax_ref[0, 0] + ax_ref[1, 0]) * disp
    for c in range(7):
        h = jnp.dot(a1p, w_ref[c], preferred_element_type=jnp.float32)
        h = h + b_ref[c]
        h = h * jax.nn.sigmoid(h)
        out_refs[c][...] = h * disp


def _tc_final_body(t0, t1, t2, t3, t4, t5, t6, ah_ref, dp_ref, bt_ref,
                   w2_ref, b2_ref, wl1_ref, bl1_ref, wl2_ref, bl2_ref,
                   out_ref, sums, cnt):
    i = pl.program_id(0)

    @pl.when(i == 0)
    def _():
        sums[...] = jnp.zeros_like(sums)
        cnt[...] = jnp.zeros_like(cnt)

    disp = _dis_packed(dp_ref[...])
    ts = (t0, t1, t2, t3, t4, t5, t6)
    h2 = b2_ref[...]
    for c in range(7):
        p_c = (ts[c][...] + ah_ref[0, c] + ah_ref[1, c]) * disp
        h2 = h2 + jnp.dot(p_c, w2_ref[c], preferred_element_type=jnp.float32)
    h2 = h2 * jax.nn.sigmoid(h2)
    gids = lax.broadcasted_iota(jnp.int32, (G, PBn), 0)
    for j in range(8):
        ohj = (gids == bt_ref[0, j, :][None, :]).astype(jnp.float32)
        sums[...] += jnp.dot(ohj, h2[:, 200 * j:200 * (j + 1)],
                             preferred_element_type=jnp.float32)
        cnt[...] += jnp.broadcast_to(
            jnp.sum(ohj, axis=1, keepdims=True), (G, 128))

    @pl.when(i == NB - 1)
    def _():
        pooled = sums[...] / jnp.maximum(cnt[:, 0:1], 1.0)
        z = jnp.dot(pooled, wl1_ref[...],
                    preferred_element_type=jnp.float32) + bl1_ref[...]
        z = z * jax.nn.sigmoid(z)
        o = jnp.dot(z, wl2_ref[...],
                    preferred_element_type=jnp.float32) + bl2_ref[...]
        out_ref[...] = o


@functools.cache
def _tc_kernels(interpret=False):
    tc_prep = pl.pallas_call(
        _tc_prep_body,
        interpret=interpret,
        grid=(NB,),
        in_specs=[
            pl.BlockSpec((2, 1, PBn, 128), lambda i: (0, 0, i, 0)),
            pl.BlockSpec((PBn, 128), lambda i: (i, 0)),
        ],
        out_specs=pl.BlockSpec((PBn, 128), lambda i: (i, 0)),
        out_shape=jax.ShapeDtypeStruct((NPP, 128), jnp.float32),
    )
    tc_mid = pl.pallas_call(
        _tc_mid_body,
        interpret=interpret,
        grid=(NB,),
        in_specs=[
            pl.BlockSpec((PBn, 128), lambda i: (i, 0)),
            pl.BlockSpec((2, 1, PBn, 128), lambda i: (0, 0, i, 0)),
            pl.BlockSpec((2, 1, PBn, 128), lambda i: (0, 0, i, 0)),
            pl.BlockSpec((7, 128, 128), lambda i: (0, 0, 0)),
            pl.BlockSpec((7, 1, 128), lambda i: (0, 0, 0)),
        ],
        out_specs=[pl.BlockSpec((PBn, 128), lambda i: (i, 0)) for _ in range(7)],
        out_shape=[jax.ShapeDtypeStruct((NPP, 128), jnp.float32) for _ in range(7)],
    )
    tc_final = pl.pallas_call(
        _tc_final_body,
        interpret=interpret,
        grid=(NB,),
        in_specs=(
            [pl.BlockSpec((PBn, 128), lambda i: (i, 0)) for _ in range(7)] + [
                pl.BlockSpec((2, 7, PBn, 128), lambda i: (0, 0, i, 0)),
                pl.BlockSpec((2, 1, PBn, 128), lambda i: (0, 0, i, 0)),
                pl.BlockSpec((1, 8, PBn), lambda i: (i, 0, 0)),
                pl.BlockSpec((7, 128, 1600), lambda i: (0, 0, 0)),
                pl.BlockSpec((1, 1600), lambda i: (0, 0)),
                pl.BlockSpec((200, 100), lambda i: (0, 0)),
                pl.BlockSpec((1, 100), lambda i: (0, 0)),
                pl.BlockSpec((100, 128), lambda i: (0, 0)),
                pl.BlockSpec((1, 128), lambda i: (0, 0)),
            ]
        ),
        out_specs=pl.BlockSpec((G, 128), lambda i: (0, 0)),
        out_shape=jax.ShapeDtypeStruct((G, 128), jnp.float32),
        scratch_shapes=[
            pltpu.VMEM((G, 200), jnp.float32),
            pltpu.VMEM((G, 128), jnp.float32),
        ],
    )
    return tc_prep, tc_mid, tc_final


def kernel(x, edge_index, batch, W1, b1, W2, b2, Wl1, bl1, Wl2, bl2):
    f32 = jnp.float32
    src = edge_index[0]
    dst = edge_index[1]
    epad = jnp.full((EPA - E,), N, jnp.int32)
    srcp = jnp.concatenate([src, epad])
    dstp = jnp.concatenate([dst, epad])
    xpad = jnp.zeros((NP, 16), f32).at[:N, :3].set(
        x.astype(f32)).reshape(NPP, 128)
    zeros16 = jnp.zeros((NP, 16), f32)
    onesv = jnp.ones((S, 16), f32)
    batchp = jnp.concatenate(
        [batch, jnp.full((NP - N,), G, jnp.int32)]).reshape(
            NB, PBn, 8).transpose(0, 2, 1)
    W1p = jnp.zeros((16, 112), f32).at[:3, :100].set(W1)
    b1p = jnp.zeros((112,), f32).at[:100].set(b1)
    eye8 = jnp.eye(8, dtype=f32)
    W1big = jnp.stack([
        jnp.einsum('ab,km->akbm', eye8,
                   W1p[:, 16 * c:16 * (c + 1)]).reshape(128, 128)
        for c in range(7)])
    b1big = jnp.stack([jnp.tile(b1p[16 * c:16 * (c + 1)], 8).reshape(1, 128)
                       for c in range(7)])
    W2p = jnp.zeros((112, 200), f32).at[:100, :].set(W2)
    W2big = jnp.stack([
        jnp.einsum('ab,km->akbm', eye8,
                   W2p[16 * c:16 * (c + 1), :]).reshape(128, 1600)
        for c in range(7)])
    b2big = jnp.tile(b2, 8).reshape(1, 1600)
    bl1r = bl1.reshape(1, 100)
    Wl2p = jnp.zeros((100, 128), f32).at[:, 0:1].set(Wl2)
    bl2p = jnp.broadcast_to(bl2.reshape(1, 1), (1, 128))

    tc_prep, tc_mid, tc_final = _tc_kernels()
    degpart = _sc_pass(1, 16, False)(dstp, onesv, zeros16)
    degp = degpart.reshape(2, 1, NPP, 128)
    xs_p = tc_prep(degp, xpad)
    xs = xs_p.reshape(NP, 16)
    aggx = _sc_pass(1, 16, True)(xs, srcp, dstp, zeros16)
    ts_p = tc_mid(xs_p, aggx.reshape(2, 1, NPP, 128), degp, W1big, b1big)
    ts = [t.reshape(NP, 16) for t in ts_p]
    aggh = _sc_pass(7, 16, True)(*ts, srcp, dstp, zeros16)
    out = tc_final(*ts_p, aggh.reshape(2, 7, NPP, 128), degp, batchp,
                   W2big, b2big, Wl1, bl1r, Wl2p, bl2p)
    return out[:, 0]
